# baseline (speedup 1.0000x reference)
import numpy as np
import jax
import jax.numpy as jnp
from jax import lax
from jax.experimental import pallas as pl
from jax.experimental.pallas import tpu as pltpu
from jax.experimental.pallas import tpu_sc as plsc

_NUM_CORES = 2
_NUM_SUBCORES = 16
_NW = _NUM_CORES * _NUM_SUBCORES
_BATCH = 128


def _scatter_body(idx_hbm, win_hbm, val_hbm, out_ref, idx_v, win_v, gval_v,
                  gsem, ssem):
    c = lax.axis_index("c")
    s = lax.axis_index("s")
    wid = s * _NUM_CORES + c
    k = idx_v.shape[0]
    pltpu.sync_copy(idx_hbm.at[wid], idx_v)
    pltpu.sync_copy(win_hbm.at[wid], win_v)
    gets = [
        pltpu.make_async_copy(
            val_hbm.at[win_v.at[np.int32(j)]],
            gval_v.at[pl.ds(j * _BATCH, _BATCH)],
            gsem,
        )
        for j in range(k)
    ]
    for cp in gets:
        cp.start()
    for cp in gets:
        cp.wait()
    puts = [
        pltpu.make_async_copy(
            gval_v.at[pl.ds(j * _BATCH, _BATCH)],
            out_ref.at[idx_v.at[np.int32(j)]],
            ssem,
        )
        for j in range(k)
    ]
    for cp in puts:
        cp.start()
    for cp in puts:
        cp.wait()


def _route(idx32):
    b = idx32.shape[0]
    pos = jnp.arange(b, dtype=jnp.int32)
    sidx, perm = lax.sort((idx32, pos), num_keys=1, is_stable=True)
    is_end = jnp.concatenate(
        [sidx[1:] != sidx[:-1], jnp.ones((1,), jnp.bool_)])
    run_end = lax.cummin(jnp.where(is_end, pos, b), axis=0, reverse=True)
    wsort = perm[run_end]
    return sidx, wsort


def kernel(input, index, value):
    m, d = input.shape
    b = index.shape[0]
    per_w = b // _NW
    k = per_w // _BATCH

    in32 = lax.bitcast_convert_type(input, jnp.int32).reshape(m, 2 * d)
    val32 = lax.bitcast_convert_type(value, jnp.int32).reshape(b, 2 * d)
    idx32 = index.astype(jnp.int32)
    sidx, wsort = _route(idx32)
    idx3d = sidx.reshape(_NW, k, _BATCH)
    win3d = wsort.reshape(_NW, k, _BATCH)

    mesh = plsc.VectorSubcoreMesh(core_axis_name="c", subcore_axis_name="s")
    scatter = pl.kernel(
        _scatter_body,
        out_type=(),
        mesh=mesh,
        compiler_params=pltpu.CompilerParams(use_tc_tiling_on_sc=False),
        scratch_types=[
            pltpu.VMEM((k, _BATCH), jnp.int32),
            pltpu.VMEM((k, _BATCH), jnp.int32),
            pltpu.VMEM((per_w, 2 * d), jnp.int32),
            pltpu.SemaphoreType.DMA,
            pltpu.SemaphoreType.DMA,
        ],
    )

    out_ref = jax.new_ref(in32)
    scatter(idx3d, win3d, val32, out_ref)
    out32 = out_ref[...].reshape(m, d, 2)
    return lax.bitcast_convert_type(out32, jnp.int64)
